# baseline (speedup 1.0000x reference)
import functools

import jax
import jax.numpy as jnp
from jax import lax
from jax.experimental import pallas as pl
from jax.experimental.pallas import tpu as pltpu
from jax.experimental.pallas import tpu_sc as plsc

N = 10000
E = 160000
NC = 2
NS = 16
NEG = 8
B = 128
NBLK = 158
EPG = NBLK * B
EPAD = NEG * EPG
HALF = 5120
JUNK = HALF
ACCR = 5184

_MESH = plsc.VectorSubcoreMesh(
    core_axis_name="c", subcore_axis_name="s", num_cores=NC, num_subcores=NS
)
_NLP = pltpu.CompilerParams(needs_layout_passes=False,
                            use_tc_tiling_on_sc=False)


def _make_agg(nch):
    npc = nch // NC

    scratch = [
        pltpu.VMEM((NBLK, B), jnp.int32),
        pltpu.VMEM((EPG,), jnp.int32),
        pltpu.VMEM((2, 16), jnp.int32),
        pltpu.VMEM((B, 16), jnp.float32),
        pltpu.VMEM((B, 16), jnp.float32),
        pltpu.VMEM((ACCR, 16), jnp.float32),
        pltpu.SemaphoreType.DMA,
        pltpu.SemaphoreType.DMA,
    ]

    def body(tab, srcs, dsth, bounds, out, src_v, dsta_v, bnd_v, rows0,
             rows1, acc, sem0, sem1):
        c = lax.axis_index("c")
        s = lax.axis_index("s")
        e = lax.rem(s, NEG)
        h = lax.div(s, NEG)

        pltpu.sync_copy(srcs.at[e], src_v)
        pltpu.sync_copy(dsth.at[h, e], dsta_v)
        pltpu.sync_copy(bounds.at[h, e], bnd_v)
        lo = jnp.max(bnd_v[0, :])
        hi = jnp.max(bnd_v[1, :])

        iota = lax.iota(jnp.int32, 16)

        def _off(delta):
            def _upd(j, carry):
                for k in range(B // 16):
                    src_v[j, pl.ds(k * 16, 16)] = (
                        src_v[j, pl.ds(k * 16, 16)] + delta
                    )
                return carry
            lax.fori_loop(0, NBLK, _upd, 0)

        _off(c * (npc * N) - N)

        zero16 = jnp.zeros((16,), jnp.int32)

        def _process(rows_b, j):
            for g in range(B // 16):
                d16 = dsta_v[pl.ds(j * B + g * 16, 16)]
                for u in range(16):
                    row = zero16 + d16[u]
                    vals = rows_b[g * 16 + u, :]
                    plsc.addupdate_scatter(acc, [row, iota], vals)

        def _chunk(cc, carry):
            chunk = c * npc + cc
            _off(N)

            def _zero(i, carry):
                acc[i, :] = jnp.zeros((16,), jnp.float32)
                return carry
            lax.fori_loop(0, ACCR, _zero, 0)

            pltpu.async_copy(tab.at[src_v.at[0]], rows0, sem0)

            def _step(t, carry):
                j = 2 * t
                pltpu.make_async_copy(tab.at[src_v.at[j]], rows0, sem0).wait()
                pltpu.async_copy(tab.at[src_v.at[j + 1]], rows1, sem1)

                @pl.when((j >= lo) & (j < hi))
                def _():
                    _process(rows0, j)

                pltpu.make_async_copy(
                    tab.at[src_v.at[j + 1]], rows1, sem1).wait()

                @pl.when(t < NBLK // 2 - 1)
                def _():
                    pltpu.async_copy(tab.at[src_v.at[j + 2]], rows0, sem0)

                @pl.when((j + 1 >= lo) & (j + 1 < hi))
                def _():
                    _process(rows1, j + 1)
                return carry

            lax.fori_loop(0, NBLK // 2, _step, 0)

            pltpu.sync_copy(
                acc, out.at[h, e, pl.ds(0, ACCR), pl.ds(chunk * 16, 16)])
            return carry

        lax.fori_loop(0, npc, _chunk, 0)

    return pl.kernel(
        body,
        out_type=jax.ShapeDtypeStruct((2, NEG, ACCR, nch * 16),
                                      jnp.float32),
        mesh=_MESH,
        compiler_params=_NLP,
        scratch_types=scratch,
    )


_agg16 = _make_agg(16)
_agg32 = _make_agg(32)
_agg2 = _make_agg(2)


def _stack16(x, nch):
    n = x.shape[0]
    return x.reshape(n, nch, 16).transpose(1, 0, 2).reshape(nch * n, 16)


_R = 640


def _merge(p_ref):
    p = p_ref[0]
    acc = p[0]
    for g in range(1, NEG):
        acc = acc + p[g]
    return acc


def _degcol(d_ref):
    p = d_ref[0]
    acc = p[0]
    for g in range(1, NEG):
        acc = acc + p[g]
    return acc[:, 0:1]


def _tc12_body(nch, w3, p_ref, d_ref, x_ref, wl_ref, wr_ref, b_ref, *rest):
    if w3:
        w3l_ref, o_ref, y3_ref = rest
    else:
        (o_ref,) = rest
    del nch
    agg = _merge(p_ref)
    deg = jnp.maximum(_degcol(d_ref), 1.0)
    hh = jnp.dot(agg / deg, wl_ref[...], preferred_element_type=jnp.float32)
    hh = hh + jnp.dot(x_ref[...], wr_ref[...],
                      preferred_element_type=jnp.float32)
    hh = hh + b_ref[...]
    nn = jnp.sqrt(jnp.sum(hh * hh, axis=1, keepdims=True))
    hh = hh / jnp.maximum(nn, 1e-12)
    hh = jnp.maximum(hh, 0.0)
    o_ref[...] = hh
    if w3:
        y3_ref[...] = jnp.dot(hh, w3l_ref[...],
                              preferred_element_type=jnp.float32)


def _tc3_body(p_ref, d_ref, x_ref, wr_ref, b_ref, o_ref):
    agg = _merge(p_ref)
    deg = jnp.maximum(_degcol(d_ref), 1.0)
    hh = agg / deg
    hh = hh + jnp.dot(x_ref[...], wr_ref[...],
                      preferred_element_type=jnp.float32)
    o_ref[...] = hh + b_ref[...]


def _p_spec(d):
    return pl.BlockSpec((1, NEG, _R, d), lambda h, i: (h, 0, i, 0))


_D_SPEC = pl.BlockSpec((1, NEG, _R, 16), lambda h, i: (h, 0, i, 0))


def _row_spec(d):
    return pl.BlockSpec((_R, d), lambda h, i: (h * (HALF // _R) + i, 0))


def _full_spec(a, b):
    return pl.BlockSpec((a, b), lambda h, i: (0, 0))


_GRID = (2, HALF // _R)


def _tc1(p, d, x, wl, wr, b):
    return pl.pallas_call(
        functools.partial(_tc12_body, 16, False),
        grid=_GRID,
        in_specs=[_p_spec(256), _D_SPEC, _row_spec(256),
                  _full_spec(256, 512), _full_spec(256, 512),
                  _full_spec(1, 512)],
        out_specs=_row_spec(512),
        out_shape=jax.ShapeDtypeStruct((2 * HALF, 512), jnp.float32),
    )(p, d, x, wl, wr, b)


def _tc2(p, d, x, wl, wr, b, w3l):
    return pl.pallas_call(
        functools.partial(_tc12_body, 32, True),
        grid=_GRID,
        in_specs=[_p_spec(512), _D_SPEC, _row_spec(512),
                  _full_spec(512, 512), _full_spec(512, 512),
                  _full_spec(1, 512), _full_spec(512, 256)],
        out_specs=[_row_spec(512), _row_spec(256)],
        out_shape=[jax.ShapeDtypeStruct((2 * HALF, 512), jnp.float32),
                   jax.ShapeDtypeStruct((2 * HALF, 256), jnp.float32)],
    )(p, d, x, wl, wr, b, w3l)


def _tc3(p, d, x, wr, b):
    return pl.pallas_call(
        _tc3_body,
        grid=_GRID,
        in_specs=[_p_spec(256), _D_SPEC, _row_spec(512),
                  _full_spec(512, 256), _full_spec(1, 256)],
        out_specs=_row_spec(256),
        out_shape=jax.ShapeDtypeStruct((2 * HALF, 256), jnp.float32),
    )(p, d, x, wr, b)


def _pad_rows(x):
    return jnp.concatenate(
        [x, jnp.zeros((2 * HALF - N, x.shape[1]), x.dtype)])


def kernel(x, edge_index, W1l, W1r, b1, W2l, W2r, b2, W3l, W3r, b3):
    src = edge_index[0]
    dst = edge_index[1]
    pad = EPAD - E
    src_pad = jnp.concatenate([src, jnp.zeros((pad,), jnp.int32)])
    dstp = jnp.concatenate([dst, jnp.full((pad,), -1, jnp.int32)])

    kg = (dstp >= HALF).reshape(NEG, EPG).astype(jnp.int32)
    c1 = jnp.cumsum(kg, axis=1)
    n0 = EPG - c1[:, -1]
    i_idx = jnp.arange(EPG, dtype=jnp.int32)[None, :]
    pos = jnp.where(kg == 0, i_idx - c1, n0[:, None] + c1 - 1)
    pos_flat = (pos + jnp.arange(NEG, dtype=jnp.int32)[:, None] * EPG).reshape(-1)
    gidx = jnp.zeros((EPAD,), jnp.int32).at[pos_flat].set(
        jnp.arange(EPAD, dtype=jnp.int32), unique_indices=True)
    src_p = jnp.take(src_pad, gidx)
    dstp = jnp.take(dstp, gidx)

    srcs = src_p.reshape(NEG, NBLK, B)
    dsth = jnp.stack([
        jnp.where((dstp >= h * HALF) & (dstp < (h + 1) * HALF),
                  dstp - h * HALF, JUNK).astype(jnp.int32)
        for h in (0, 1)]).reshape(2, NEG, EPG)
    hi0 = (n0 + B - 1) // B
    lo1 = n0 // B
    bounds = jnp.stack([
        jnp.stack([jnp.zeros_like(n0), hi0], axis=1),
        jnp.stack([lo1, jnp.full_like(n0, NBLK)], axis=1),
    ])
    bounds = (jnp.broadcast_to(bounds[..., None], (2, NEG, 2, 16))
              .astype(jnp.int32) + 0)

    dp = _agg2(jnp.ones((2 * N, 16), jnp.float32), srcs, dsth, bounds)
    d4 = dp[:, :, :, :16]

    p1 = _agg16(_stack16(x, 16), srcs, dsth, bounds)
    h1 = _tc1(p1, d4, _pad_rows(x), W1l, W1r, b1.reshape(1, -1))

    p2 = _agg32(_stack16(h1[:N], 32), srcs, dsth, bounds)
    h2, y3 = _tc2(p2, d4, h1, W2l, W2r, b2.reshape(1, -1), W3l)

    p3 = _agg16(_stack16(y3[:N], 16), srcs, dsth, bounds)
    out = _tc3(p3, d4, h2, W3r, b3.reshape(1, -1))
    return out[:N]


# speedup vs baseline: 1.4734x; 1.4734x over previous
import functools

import jax
import jax.numpy as jnp
from jax import lax
from jax.experimental import pallas as pl
from jax.experimental.pallas import tpu as pltpu
from jax.experimental.pallas import tpu_sc as plsc

N = 10000
E = 160000
NC = 2
NS = 16
NEG = 8
B = 128
NBLK = 158
EPG = NBLK * B
EPAD = NEG * EPG
HALF = 5120
JUNK = HALF
ACCR = 5184

_MESH = plsc.VectorSubcoreMesh(
    core_axis_name="c", subcore_axis_name="s", num_cores=NC, num_subcores=NS
)
_NLP = pltpu.CompilerParams(needs_layout_passes=False,
                            use_tc_tiling_on_sc=False)


def _make_agg(nch):
    npc = nch // NC

    scratch = [
        pltpu.VMEM((NBLK, B), jnp.int32),
        pltpu.VMEM((EPG,), jnp.int32),
        pltpu.VMEM((2, 16), jnp.int32),
        pltpu.VMEM((B, 16), jnp.float32),
        pltpu.VMEM((B, 16), jnp.float32),
        pltpu.VMEM((ACCR, 16), jnp.float32),
        pltpu.SemaphoreType.DMA,
        pltpu.SemaphoreType.DMA,
    ]

    def body(tab, srcs, dsth, bounds, out, src_v, dsta_v, bnd_v, rows0,
             rows1, acc, sem0, sem1):
        c = lax.axis_index("c")
        s = lax.axis_index("s")
        e = lax.rem(s, NEG)
        h = lax.div(s, NEG)

        pltpu.sync_copy(srcs.at[e], src_v)
        pltpu.sync_copy(dsth.at[h, e], dsta_v)
        pltpu.sync_copy(bounds.at[h, e], bnd_v)
        lo = jnp.max(bnd_v[0, :])
        hi = jnp.max(bnd_v[1, :])

        iota = lax.iota(jnp.int32, 16)

        def _off(delta):
            def _upd(j, carry):
                for k in range(B // 16):
                    src_v[j, pl.ds(k * 16, 16)] = (
                        src_v[j, pl.ds(k * 16, 16)] + delta
                    )
                return carry
            lax.fori_loop(0, NBLK, _upd, 0)

        _off(c * (npc * N) - N)

        zero16 = jnp.zeros((16,), jnp.int32)

        def _process(rows_b, j):
            for g in range(B // 16):
                d16 = dsta_v[pl.ds(j * B + g * 16, 16)]
                for u in range(16):
                    row = zero16 + d16[u]
                    vals = rows_b[g * 16 + u, :]
                    plsc.addupdate_scatter(acc, [row, iota], vals)

        def _chunk(cc, carry):
            chunk = c * npc + cc
            _off(N)

            def _zero(i, carry):
                acc[i, :] = jnp.zeros((16,), jnp.float32)
                return carry
            lax.fori_loop(0, ACCR, _zero, 0)

            def _cond(b):
                return (b >= lo) & (b < hi) & (b < NBLK)

            @pl.when(_cond(0))
            def _():
                pltpu.async_copy(tab.at[src_v.at[0]], rows0, sem0)

            @pl.when(_cond(1))
            def _():
                pltpu.async_copy(tab.at[src_v.at[1]], rows1, sem1)

            def _step(t, carry):
                j = 2 * t

                @pl.when(_cond(j))
                def _():
                    pltpu.make_async_copy(
                        tab.at[src_v.at[j]], rows0, sem0).wait()
                    _process(rows0, j)

                @pl.when(_cond(j + 2))
                def _():
                    pltpu.async_copy(tab.at[src_v.at[j + 2]], rows0, sem0)

                @pl.when(_cond(j + 1))
                def _():
                    pltpu.make_async_copy(
                        tab.at[src_v.at[j + 1]], rows1, sem1).wait()
                    _process(rows1, j + 1)

                @pl.when(_cond(j + 3))
                def _():
                    pltpu.async_copy(tab.at[src_v.at[j + 3]], rows1, sem1)
                return carry

            lax.fori_loop(0, NBLK // 2, _step, 0)

            pltpu.sync_copy(
                acc, out.at[h, e, pl.ds(0, ACCR), pl.ds(chunk * 16, 16)])
            return carry

        lax.fori_loop(0, npc, _chunk, 0)

    return pl.kernel(
        body,
        out_type=jax.ShapeDtypeStruct((2, NEG, ACCR, nch * 16),
                                      jnp.float32),
        mesh=_MESH,
        compiler_params=_NLP,
        scratch_types=scratch,
    )


_agg16 = _make_agg(16)
_agg32 = _make_agg(32)
_agg2 = _make_agg(2)


def _stack16(x, nch):
    n = x.shape[0]
    return x.reshape(n, nch, 16).transpose(1, 0, 2).reshape(nch * n, 16)


_R = 640


def _merge(p_ref):
    p = p_ref[0]
    acc = p[0]
    for g in range(1, NEG):
        acc = acc + p[g]
    return acc


def _degcol(d_ref):
    p = d_ref[0]
    acc = p[0]
    for g in range(1, NEG):
        acc = acc + p[g]
    return acc[:, 0:1]


def _tc12_body(nch, w3, p_ref, d_ref, x_ref, wl_ref, wr_ref, b_ref, *rest):
    if w3:
        w3l_ref, o_ref, y3_ref = rest
    else:
        (o_ref,) = rest
    del nch
    agg = _merge(p_ref)
    deg = jnp.maximum(_degcol(d_ref), 1.0)
    hh = jnp.dot(agg / deg, wl_ref[...], preferred_element_type=jnp.float32)
    hh = hh + jnp.dot(x_ref[...], wr_ref[...],
                      preferred_element_type=jnp.float32)
    hh = hh + b_ref[...]
    nn = jnp.sqrt(jnp.sum(hh * hh, axis=1, keepdims=True))
    hh = hh / jnp.maximum(nn, 1e-12)
    hh = jnp.maximum(hh, 0.0)
    o_ref[...] = hh
    if w3:
        y3_ref[...] = jnp.dot(hh, w3l_ref[...],
                              preferred_element_type=jnp.float32)


def _tc3_body(p_ref, d_ref, x_ref, wr_ref, b_ref, o_ref):
    agg = _merge(p_ref)
    deg = jnp.maximum(_degcol(d_ref), 1.0)
    hh = agg / deg
    hh = hh + jnp.dot(x_ref[...], wr_ref[...],
                      preferred_element_type=jnp.float32)
    o_ref[...] = hh + b_ref[...]


def _p_spec(d):
    return pl.BlockSpec((1, NEG, _R, d), lambda h, i: (h, 0, i, 0))


_D_SPEC = pl.BlockSpec((1, NEG, _R, 16), lambda h, i: (h, 0, i, 0))


def _row_spec(d):
    return pl.BlockSpec((_R, d), lambda h, i: (h * (HALF // _R) + i, 0))


def _full_spec(a, b):
    return pl.BlockSpec((a, b), lambda h, i: (0, 0))


_GRID = (2, HALF // _R)


def _tc1(p, d, x, wl, wr, b):
    return pl.pallas_call(
        functools.partial(_tc12_body, 16, False),
        grid=_GRID,
        in_specs=[_p_spec(256), _D_SPEC, _row_spec(256),
                  _full_spec(256, 512), _full_spec(256, 512),
                  _full_spec(1, 512)],
        out_specs=_row_spec(512),
        out_shape=jax.ShapeDtypeStruct((2 * HALF, 512), jnp.float32),
    )(p, d, x, wl, wr, b)


def _tc2(p, d, x, wl, wr, b, w3l):
    return pl.pallas_call(
        functools.partial(_tc12_body, 32, True),
        grid=_GRID,
        in_specs=[_p_spec(512), _D_SPEC, _row_spec(512),
                  _full_spec(512, 512), _full_spec(512, 512),
                  _full_spec(1, 512), _full_spec(512, 256)],
        out_specs=[_row_spec(512), _row_spec(256)],
        out_shape=[jax.ShapeDtypeStruct((2 * HALF, 512), jnp.float32),
                   jax.ShapeDtypeStruct((2 * HALF, 256), jnp.float32)],
    )(p, d, x, wl, wr, b, w3l)


def _tc3(p, d, x, wr, b):
    return pl.pallas_call(
        _tc3_body,
        grid=_GRID,
        in_specs=[_p_spec(256), _D_SPEC, _row_spec(512),
                  _full_spec(512, 256), _full_spec(1, 256)],
        out_specs=_row_spec(256),
        out_shape=jax.ShapeDtypeStruct((2 * HALF, 256), jnp.float32),
    )(p, d, x, wr, b)


def _pad_rows(x):
    return jnp.concatenate(
        [x, jnp.zeros((2 * HALF - N, x.shape[1]), x.dtype)])


def kernel(x, edge_index, W1l, W1r, b1, W2l, W2r, b2, W3l, W3r, b3):
    src = edge_index[0]
    dst = edge_index[1]
    pad = EPAD - E
    src_pad = jnp.concatenate([src, jnp.zeros((pad,), jnp.int32)])
    dstp = jnp.concatenate([dst, jnp.full((pad,), -1, jnp.int32)])

    kg = (dstp >= HALF).reshape(NEG, EPG).astype(jnp.int32)
    c1 = jnp.cumsum(kg, axis=1)
    n0 = EPG - c1[:, -1]
    i_idx = jnp.arange(EPG, dtype=jnp.int32)[None, :]
    pos = jnp.where(kg == 0, i_idx - c1, n0[:, None] + c1 - 1)
    pos_flat = (pos + jnp.arange(NEG, dtype=jnp.int32)[:, None] * EPG).reshape(-1)
    gidx = jnp.zeros((EPAD,), jnp.int32).at[pos_flat].set(
        jnp.arange(EPAD, dtype=jnp.int32), unique_indices=True)
    src_p = jnp.take(src_pad, gidx)
    dstp = jnp.take(dstp, gidx)

    srcs = src_p.reshape(NEG, NBLK, B)
    dsth = jnp.stack([
        jnp.where((dstp >= h * HALF) & (dstp < (h + 1) * HALF),
                  dstp - h * HALF, JUNK).astype(jnp.int32)
        for h in (0, 1)]).reshape(2, NEG, EPG)
    hi0 = (n0 + B - 1) // B
    lo1 = n0 // B
    bounds = jnp.stack([
        jnp.stack([jnp.zeros_like(n0), hi0], axis=1),
        jnp.stack([lo1, jnp.full_like(n0, NBLK)], axis=1),
    ])
    bounds = (jnp.broadcast_to(bounds[..., None], (2, NEG, 2, 16))
              .astype(jnp.int32) + 0)

    dp = _agg2(jnp.ones((2 * N, 16), jnp.float32), srcs, dsth, bounds)
    d4 = dp[:, :, :, :16]

    p1 = _agg16(_stack16(x, 16), srcs, dsth, bounds)
    h1 = _tc1(p1, d4, _pad_rows(x), W1l, W1r, b1.reshape(1, -1))

    p2 = _agg32(_stack16(h1[:N], 32), srcs, dsth, bounds)
    h2, y3 = _tc2(p2, d4, h1, W2l, W2r, b2.reshape(1, -1), W3l)

    p3 = _agg16(_stack16(y3[:N], 16), srcs, dsth, bounds)
    out = _tc3(p3, d4, h2, W3r, b3.reshape(1, -1))
    return out[:N]
